# Initial kernel scaffold; baseline (speedup 1.0000x reference)
import jax
import jax.numpy as jnp
from jax.experimental import pallas as pl


def kernel(x, abs_x, idx, k, v, Wq, Wk, Wv, Wmq, Wmk, Wmv, Wm2nl, bn_gamma, bn_beta):
    raise NotImplementedError("write your pallas kernel here")


# speedup vs baseline: 4.9293x; 4.9293x over previous
import functools

import jax
import jax.numpy as jnp
from jax import lax
from jax.experimental import pallas as pl

B, CIN, N, K = 2, 256, 2048, 16
G, LCH, MCH, NLCH = 4, 192, 64, 64
CG = LCH // G
MG = MCH // G

TN = 128
NT = N // TN

_bf16 = jnp.bfloat16
_f32 = jnp.float32


def _dot(a, b, precision=None):
    return jax.lax.dot(a, b, precision=precision, preferred_element_type=_f32)


def _stage1_body(idx_ref, x_ref, wq_ref, wk_ref, wv_ref, out_local_ref, score_ref):
    t = pl.program_id(1)
    x = x_ref[0].reshape(CIN, K * TN)
    xb = x.astype(_bf16)
    q = _dot(wq_ref[...].astype(_bf16), xb)
    k = _dot(wk_ref[...].astype(_bf16), xb)
    v = _dot(wv_ref[...].astype(_bf16), xb)

    s = (q * k).reshape(G, CG, K * TN).sum(axis=1)
    s = s.reshape(G, K, TN)
    m = s.max(axis=1, keepdims=True)
    e = jnp.exp(s - m)
    attn = e / e.sum(axis=1, keepdims=True)

    vg = v.reshape(G, CG, K, TN)
    ol = (vg * attn[:, None]).sum(axis=2)
    out_local_ref[0] = ol.reshape(LCH, TN)

    idxb = idx_ref[0]
    eq = idxb[None, :, :] == idxb[:, None, :]
    ki = lax.broadcasted_iota(jnp.int32, (K, K, TN), 0)
    ji = lax.broadcasted_iota(jnp.int32, (K, K, TN), 1)
    dup = jnp.any(eq & (ji > ki), axis=1)
    am = attn * jnp.where(dup, 0.0, 1.0)[None]

    @pl.when(t == 0)
    def _():
        score_ref[...] = jnp.zeros_like(score_ref)

    iota_n = lax.broadcasted_iota(jnp.int32, (TN, N), 1)
    acc = jnp.zeros((G, N), _f32)
    for kk in range(K):
        oh = (idxb[kk][:, None] == iota_n).astype(_f32)
        acc = acc + _dot(am[:, kk, :], oh, precision=lax.Precision.HIGHEST)
    score_ref[0] += acc


def _stage2_body(score_ref, absx_ref, wmq_ref, wmk_ref, wmv_ref, wm2nl_ref,
                 gamma_ref, beta_ref, out_all_ref, kmo_ref, vmo_ref):
    sc = score_ref[...].reshape(B * G, N)
    iota = lax.broadcasted_iota(jnp.int32, (B * G, N), 1)
    work = sc
    vals, idxs = [], []
    for _ in range(K):
        mv = work.max(axis=1, keepdims=True)
        cand = jnp.where(work == mv, iota, N)
        ti = cand.min(axis=1, keepdims=True)
        vals.append(mv)
        idxs.append(ti)
        work = jnp.where(iota == ti, -jnp.inf, work)
    top_val = jnp.concatenate(vals, axis=1)
    top_idx = jnp.concatenate(idxs, axis=1)
    gate = jnp.tanh(top_val)

    iota_sel = lax.broadcasted_iota(jnp.int32, (N, K), 0)
    wmq = wmq_ref[...].astype(_bf16)
    wmk = wmk_ref[...].astype(_bf16)
    wmv = wmv_ref[...].astype(_bf16)

    for b in range(B):
        ax = absx_ref[b].astype(_bf16)
        qm = _dot(wmq, ax)
        km = _dot(wmk, ax)
        vm = _dot(wmv, ax)
        rows = []
        for g in range(G):
            r = b * G + g
            oh = (iota_sel == top_idx[r][None, :]).astype(_f32)
            kmem = _dot(km[g * MG:(g + 1) * MG], oh,
                        precision=lax.Precision.HIGHEST)
            vmem = _dot(vm[g * MG:(g + 1) * MG], oh,
                        precision=lax.Precision.HIGHEST)
            vmem = vmem * gate[r][None, :]
            kmo_ref[b, g] = kmem
            vmo_ref[b, g] = vmem
            at = lax.dot_general(
                kmem.astype(_bf16), qm[g * MG:(g + 1) * MG].astype(_bf16),
                (((0,), (0,)), ((), ())), preferred_element_type=_f32)
            at = at - at.max(axis=0, keepdims=True)
            ea = jnp.exp(at)
            aw = ea / ea.sum(axis=0, keepdims=True)
            rows.append(_dot(vmem.astype(_bf16), aw.astype(_bf16)))
        out_pre = jnp.concatenate(rows, axis=0)
        out_all_ref[b] = _dot(wm2nl_ref[...].astype(_bf16), out_pre.astype(_bf16))

    oa = out_all_ref[...]
    mean = oa.sum(axis=(0, 2), keepdims=True) / (B * N)
    d = oa - mean
    var = (d * d).sum(axis=(0, 2), keepdims=True) / (B * N)
    out = d * lax.rsqrt(var + 1e-5)
    out_all_ref[...] = out * gamma_ref[...][None, :, :] + beta_ref[...][None, :, :]


@jax.jit
def _run(x, abs_x, idx, Wq, Wk, Wv, Wmq, Wmk, Wmv, Wm2nl, bn_gamma, bn_beta):
    xt = x.transpose(0, 1, 3, 2)
    idxt = idx.reshape(B, N, K).transpose(0, 2, 1)

    out_local, score = pl.pallas_call(
        _stage1_body,
        grid=(B, NT),
        in_specs=[
            pl.BlockSpec((1, K, TN), lambda b, t: (b, 0, t)),
            pl.BlockSpec((1, CIN, K, TN), lambda b, t: (b, 0, 0, t)),
            pl.BlockSpec((LCH, CIN), lambda b, t: (0, 0)),
            pl.BlockSpec((LCH, CIN), lambda b, t: (0, 0)),
            pl.BlockSpec((LCH, CIN), lambda b, t: (0, 0)),
        ],
        out_specs=[
            pl.BlockSpec((1, LCH, TN), lambda b, t: (b, 0, t)),
            pl.BlockSpec((1, G, N), lambda b, t: (b, 0, 0)),
        ],
        out_shape=[
            jax.ShapeDtypeStruct((B, LCH, N), _f32),
            jax.ShapeDtypeStruct((B, G, N), _f32),
        ],
    )(idxt, xt, Wq, Wk, Wv)

    out_all, kmo, vmo = pl.pallas_call(
        _stage2_body,
        out_shape=[
            jax.ShapeDtypeStruct((B, NLCH, N), _f32),
            jax.ShapeDtypeStruct((B, G, MG, K), _f32),
            jax.ShapeDtypeStruct((B, G, MG, K), _f32),
        ],
    )(score, abs_x.reshape(B, CIN // 2, N), Wmq, Wmk, Wmv, Wm2nl,
      bn_gamma.reshape(NLCH, 1), bn_beta.reshape(NLCH, 1))

    out_final = jnp.concatenate(
        [out_local.reshape(B, LCH, N, 1), out_all.reshape(B, NLCH, N, 1)], axis=1)
    return out_final, kmo, vmo


def kernel(x, abs_x, idx, k, v, Wq, Wk, Wv, Wmq, Wmk, Wmv, Wm2nl, bn_gamma, bn_beta):
    del k, v
    return _run(x, abs_x, idx, Wq, Wk, Wv, Wmq, Wmk, Wmv, Wm2nl, bn_gamma, bn_beta)
